# baseline (speedup 1.0000x reference)
import jax
import jax.numpy as jnp
from jax.experimental import pallas as pl
from jax.experimental.pallas import tpu as pltpu

_PAIRS = 64


def _pool_body(x_ref, o_ref):
    for b in range(_PAIRS):
        base = 448 * b
        v00 = x_ref[pl.ds(base + 0, 112, 2), :]
        v01 = x_ref[pl.ds(base + 1, 112, 2), :]
        v10 = x_ref[pl.ds(base + 224, 112, 2), :]
        v11 = x_ref[pl.ds(base + 225, 112, 2), :]
        o_ref[pl.ds(112 * b, 112), :] = jnp.maximum(
            jnp.maximum(v00, v01), jnp.maximum(v10, v11)
        )


def kernel(x):
    n, c, hh, ww = x.shape
    xt = jnp.transpose(x, (0, 2, 3, 1))
    x2 = xt.reshape(n * hh * ww, c)
    rows_in = 2 * ww * _PAIRS
    grid = (n * hh * ww) // rows_in
    out = pl.pallas_call(
        _pool_body,
        grid=(grid,),
        in_specs=[pl.BlockSpec((rows_in, c), lambda i: (i, 0))],
        out_specs=pl.BlockSpec((112 * _PAIRS, c), lambda i: (i, 0)),
        out_shape=jax.ShapeDtypeStruct((n * (hh // 2) * (ww // 2), c), x.dtype),
        compiler_params=pltpu.CompilerParams(
            dimension_semantics=("parallel",),
        ),
    )(x2)
    out4 = out.reshape(n, hh // 2, ww // 2, c)
    return jnp.transpose(out4, (0, 3, 1, 2))
